# Initial kernel scaffold; baseline (speedup 1.0000x reference)
import jax
import jax.numpy as jnp
from jax.experimental import pallas as pl


def kernel(inputs, coordinates):
    raise NotImplementedError("write your pallas kernel here")


# speedup vs baseline: 3.6591x; 3.6591x over previous
import functools

import jax
import jax.numpy as jnp
from jax import lax
from jax.experimental import pallas as pl
from jax.experimental.pallas import tpu as pltpu
from jax.experimental.pallas import tpu_sc as plsc

F = 8192
D = 256
K = 16
R = 256


def _topk_body(sqc_ref, sqr_ref, call_ref, crows_ref, idx_ref):
    c = lax.dot_general(crows_ref[...], call_ref[...],
                        (((0,), (0,)), ((), ())))
    d = (sqc_ref[...] + sqr_ref[...]) - 2.0 * c
    d = jnp.maximum(d, 0.0)
    iota = lax.broadcasted_iota(jnp.int32, (R, F), 1)
    col16 = lax.broadcasted_iota(jnp.int32, (R, K), 1)

    def body(j, carry):
        dd, acc = carry
        m = jnp.min(dd, axis=1, keepdims=True)
        cand = jnp.where(dd == m, iota, jnp.int32(F))
        amin = jnp.min(cand, axis=1, keepdims=True)
        acc = jnp.where(col16 == j, amin, acc)
        dd = jnp.where(iota == amin, jnp.float32(jnp.inf), dd)
        return dd, acc

    _, acc = lax.fori_loop(0, K, body, (d, jnp.zeros((R, K), jnp.int32)))
    idx_ref[...] = acc


def _topk_indices(c, sq):
    sq_cols = sq[None, :]
    sq_rows = sq[:, None]
    return pl.pallas_call(
        _topk_body,
        grid=(F // R,),
        in_specs=[
            pl.BlockSpec((1, F), lambda i: (0, 0)),
            pl.BlockSpec((R, 1), lambda i: (i, 0)),
            pl.BlockSpec((D, F), lambda i: (0, 0)),
            pl.BlockSpec((D, R), lambda i: (0, i)),
        ],
        out_specs=pl.BlockSpec((R, K), lambda i: (i, 0)),
        out_shape=jax.ShapeDtypeStruct((F, K), jnp.int32),
    )(sq_cols, sq_rows, c, c)


def _make_gather(n_rows, n_cols, n_idx):
    info = plsc.get_sparse_core_info()
    nw = info.num_cores * info.num_subcores
    per_w = n_idx // nw
    ch = 128
    n_ch = per_w // ch
    mesh = plsc.VectorSubcoreMesh(core_axis_name="c", subcore_axis_name="s")

    @functools.partial(
        pl.kernel, mesh=mesh,
        compiler_params=pltpu.CompilerParams(use_tc_tiling_on_sc=False),
        out_type=jax.ShapeDtypeStruct((n_idx, n_cols), jnp.float32),
        scratch_types=[
            pltpu.VMEM((per_w,), jnp.int32),
            pltpu.VMEM((ch, n_cols), jnp.float32),
            pltpu.SemaphoreType.DMA,
        ],
    )
    def k(table_hbm, idx_hbm, out_hbm, idx_v, rows_v, sem):
        wid = lax.axis_index("s") * info.num_cores + lax.axis_index("c")
        base = wid * per_w
        pltpu.sync_copy(idx_hbm.at[pl.ds(base, per_w)], idx_v)

        def body(j, carry):
            off = pl.multiple_of(j * ch, ch)
            pltpu.async_copy(
                table_hbm.at[idx_v.at[pl.ds(off, ch)]], rows_v, sem).wait()
            pltpu.sync_copy(rows_v, out_hbm.at[pl.ds(base + off, ch)])
            return carry

        lax.fori_loop(0, n_ch, body, 0)

    return k


def kernel(inputs, coordinates):
    b = inputs.shape[0]
    c = coordinates[:, :, 0]
    sq = jnp.sum(c * c, axis=0)
    idx = _topk_indices(c, sq)
    data = jnp.transpose(inputs[:, :, 0], (1, 0))
    gathered = _make_gather(F, b, F * K)(data, idx.reshape(-1))
    out = jnp.transpose(gathered, (1, 0)).reshape(b, F * K, 1)
    return out


# speedup vs baseline: 5.4818x; 1.4981x over previous
import functools

import jax
import jax.numpy as jnp
from jax import lax
from jax.experimental import pallas as pl
from jax.experimental.pallas import tpu as pltpu
from jax.experimental.pallas import tpu_sc as plsc

F = 8192
D = 256
K = 16
R = 256
BW = 32
NB = F // BW
KB = 20
W = KB * BW


def _dist_blocks_body(sqc_ref, sqr_ref, call_ref, crows_ref, d_ref, ridx_ref):
    c = lax.dot_general(crows_ref[...], call_ref[...],
                        (((0,), (0,)), ((), ())))
    d = (sqc_ref[...] + sqr_ref[...]) - 2.0 * c
    d = jnp.maximum(d, 0.0)
    d_ref[...] = d

    bmin = jnp.min(d.reshape(R, NB, BW), axis=2)
    iota = lax.broadcasted_iota(jnp.int32, (R, NB), 1)
    colk = lax.broadcasted_iota(jnp.int32, (R, KB), 1)

    def body(j, carry):
        bb, acc = carry
        m = jnp.min(bb, axis=1, keepdims=True)
        cand = jnp.where(bb == m, iota, jnp.int32(NB))
        amin = jnp.min(cand, axis=1, keepdims=True)
        acc = jnp.where(colk == j, amin, acc)
        bb = jnp.where(iota == amin, jnp.float32(jnp.inf), bb)
        return bb, acc

    _, blk = lax.fori_loop(0, KB, body, (bmin, jnp.zeros((R, KB), jnp.int32)))
    rows = (lax.broadcasted_iota(jnp.int32, (R, KB), 0)
            + pl.program_id(0) * R)
    ridx_ref[...] = rows * NB + blk


def _dist_and_blocks(c, sq):
    return pl.pallas_call(
        _dist_blocks_body,
        grid=(F // R,),
        in_specs=[
            pl.BlockSpec((1, F), lambda i: (0, 0)),
            pl.BlockSpec((R, 1), lambda i: (i, 0)),
            pl.BlockSpec((D, F), lambda i: (0, 0)),
            pl.BlockSpec((D, R), lambda i: (0, i)),
        ],
        out_specs=[
            pl.BlockSpec((R, F), lambda i: (i, 0)),
            pl.BlockSpec((R, KB), lambda i: (i, 0)),
        ],
        out_shape=[
            jax.ShapeDtypeStruct((F, F), jnp.float32),
            jax.ShapeDtypeStruct((F, KB), jnp.int32),
        ],
    )(sq[None, :], sq[:, None], c, c)


def _select_body(cand_ref, gidx_ref, idx_ref):
    dd0 = cand_ref[...]
    gg = gidx_ref[...]
    colk = lax.broadcasted_iota(jnp.int32, (R, K), 1)

    def body(j, carry):
        dd, acc = carry
        m = jnp.min(dd, axis=1, keepdims=True)
        ag = jnp.where(dd == m, gg, jnp.int32(F))
        amin = jnp.min(ag, axis=1, keepdims=True)
        acc = jnp.where(colk == j, amin, acc)
        dd = jnp.where(gg == amin, jnp.float32(jnp.inf), dd)
        return dd, acc

    _, acc = lax.fori_loop(0, K, body, (dd0, jnp.zeros((R, K), jnp.int32)))
    idx_ref[...] = acc


def _select_topk(cand, gidx):
    return pl.pallas_call(
        _select_body,
        grid=(F // R,),
        in_specs=[
            pl.BlockSpec((R, W), lambda i: (i, 0)),
            pl.BlockSpec((R, W), lambda i: (i, 0)),
        ],
        out_specs=pl.BlockSpec((R, K), lambda i: (i, 0)),
        out_shape=jax.ShapeDtypeStruct((F, K), jnp.int32),
    )(cand, gidx)


def _make_gather(n_rows, n_cols, n_idx, dtype):
    info = plsc.get_sparse_core_info()
    nw = info.num_cores * info.num_subcores
    per_w = n_idx // nw
    ch = 128
    n_ch = per_w // ch
    mesh = plsc.VectorSubcoreMesh(core_axis_name="c", subcore_axis_name="s")

    @functools.partial(
        pl.kernel, mesh=mesh,
        compiler_params=pltpu.CompilerParams(use_tc_tiling_on_sc=False),
        out_type=jax.ShapeDtypeStruct((n_idx, n_cols), dtype),
        scratch_types=[
            pltpu.VMEM((per_w,), jnp.int32),
            pltpu.VMEM((ch, n_cols), dtype),
            pltpu.SemaphoreType.DMA,
        ],
    )
    def k(table_hbm, idx_hbm, out_hbm, idx_v, rows_v, sem):
        wid = lax.axis_index("s") * info.num_cores + lax.axis_index("c")
        base = wid * per_w
        pltpu.sync_copy(idx_hbm.at[pl.ds(base, per_w)], idx_v)

        def body(j, carry):
            off = pl.multiple_of(j * ch, ch)
            pltpu.async_copy(
                table_hbm.at[idx_v.at[pl.ds(off, ch)]], rows_v, sem).wait()
            pltpu.sync_copy(rows_v, out_hbm.at[pl.ds(base + off, ch)])
            return carry

        lax.fori_loop(0, n_ch, body, 0)

    return k


def kernel(inputs, coordinates):
    b = inputs.shape[0]
    c = coordinates[:, :, 0]
    sq = jnp.sum(c * c, axis=0)
    d, ridx = _dist_and_blocks(c, sq)

    flat_ridx = ridx.reshape(-1)
    cand = _make_gather(F * NB, BW, F * KB, jnp.float32)(
        d.reshape(F * NB, BW), flat_ridx)
    col_table = jnp.arange(F, dtype=jnp.int32).reshape(NB, BW)
    gidx = _make_gather(NB, BW, F * KB, jnp.int32)(
        col_table, flat_ridx % NB)

    idx = _select_topk(cand.reshape(F, W), gidx.reshape(F, W))

    data = jnp.transpose(inputs[:, :, 0], (1, 0))
    gathered = _make_gather(F, b, F * K, jnp.float32)(
        data, idx.reshape(-1))
    out = jnp.transpose(gathered, (1, 0)).reshape(b, F * K, 1)
    return out


# speedup vs baseline: 8.5500x; 1.5597x over previous
import functools

import jax
import jax.numpy as jnp
from jax import lax
from jax.experimental import pallas as pl
from jax.experimental.pallas import tpu as pltpu
from jax.experimental.pallas import tpu_sc as plsc

F = 8192
D = 256
K = 16
R = 256
BW = 32
NB = F // BW
KB = 20
W = KB * BW


def _dist_blocks_body(sqc_ref, sqr_ref, call_ref, crows_ref, d_ref, blk_ref):
    c = lax.dot_general(crows_ref[...], call_ref[...],
                        (((0,), (0,)), ((), ())))
    d = (sqc_ref[...] + sqr_ref[...]) - 2.0 * c
    d = jnp.maximum(d, 0.0)
    d_ref[...] = d

    dt = jnp.transpose(d, (1, 0))
    bmin = jnp.min(dt.reshape(NB, BW, R), axis=1)
    iota_b = lax.broadcasted_iota(jnp.int32, (NB, R), 0)
    row_k = lax.broadcasted_iota(jnp.int32, (KB, R), 0)

    def body(j, carry):
        bb, acc = carry
        m = jnp.min(bb, axis=0, keepdims=True)
        cand = jnp.where(bb == m, iota_b, jnp.int32(NB))
        amin = jnp.min(cand, axis=0, keepdims=True)
        acc = jnp.where(row_k == j, amin, acc)
        bb = jnp.where(iota_b == amin, jnp.float32(jnp.inf), bb)
        return bb, acc

    _, blk_t = lax.fori_loop(0, KB, body,
                             (bmin, jnp.zeros((KB, R), jnp.int32)))
    blk_ref[...] = jnp.transpose(blk_t, (1, 0))


def _dist_and_blocks(c, sq):
    return pl.pallas_call(
        _dist_blocks_body,
        grid=(F // R,),
        in_specs=[
            pl.BlockSpec((1, F), lambda i: (0, 0)),
            pl.BlockSpec((R, 1), lambda i: (i, 0)),
            pl.BlockSpec((D, F), lambda i: (0, 0)),
            pl.BlockSpec((D, R), lambda i: (0, i)),
        ],
        out_specs=[
            pl.BlockSpec((R, F), lambda i: (i, 0)),
            pl.BlockSpec((R, KB), lambda i: (i, 0)),
        ],
        out_shape=[
            jax.ShapeDtypeStruct((F, F), jnp.float32),
            jax.ShapeDtypeStruct((F, KB), jnp.int32),
        ],
    )(sq[None, :], sq[:, None], c, c)


def _select_body(cand_ref, blk_ref, idx_ref):
    dd0 = cand_ref[...]
    blk = blk_ref[...]
    off = lax.broadcasted_iota(jnp.int32, (R, KB, BW), 2)
    gg = (blk[:, :, None] * BW + off).reshape(R, W)
    colk = lax.broadcasted_iota(jnp.int32, (R, K), 1)

    def body(j, carry):
        dd, acc = carry
        m = jnp.min(dd, axis=1, keepdims=True)
        ag = jnp.where(dd == m, gg, jnp.int32(F))
        amin = jnp.min(ag, axis=1, keepdims=True)
        acc = jnp.where(colk == j, amin, acc)
        dd = jnp.where(gg == amin, jnp.float32(jnp.inf), dd)
        return dd, acc

    _, acc = lax.fori_loop(0, K, body, (dd0, jnp.zeros((R, K), jnp.int32)))
    idx_ref[...] = acc


def _select_topk(cand, blk):
    return pl.pallas_call(
        _select_body,
        grid=(F // R,),
        in_specs=[
            pl.BlockSpec((R, W), lambda i: (i, 0)),
            pl.BlockSpec((R, KB), lambda i: (i, 0)),
        ],
        out_specs=pl.BlockSpec((R, K), lambda i: (i, 0)),
        out_shape=jax.ShapeDtypeStruct((F, K), jnp.int32),
    )(cand, blk)


def _make_gather(n_rows, n_cols, n_idx, dtype):
    info = plsc.get_sparse_core_info()
    nw = info.num_cores * info.num_subcores
    per_w = n_idx // nw
    ch = 128
    n_ch = per_w // ch
    mesh = plsc.VectorSubcoreMesh(core_axis_name="c", subcore_axis_name="s")

    @functools.partial(
        pl.kernel, mesh=mesh,
        compiler_params=pltpu.CompilerParams(use_tc_tiling_on_sc=False),
        out_type=jax.ShapeDtypeStruct((n_idx, n_cols), dtype),
        scratch_types=[
            pltpu.VMEM((per_w,), jnp.int32),
            pltpu.VMEM((ch, n_cols), dtype),
            pltpu.SemaphoreType.DMA,
        ],
    )
    def k(table_hbm, idx_hbm, out_hbm, idx_v, rows_v, sem):
        wid = lax.axis_index("s") * info.num_cores + lax.axis_index("c")
        base = wid * per_w
        pltpu.sync_copy(idx_hbm.at[pl.ds(base, per_w)], idx_v)

        def body(j, carry):
            off = pl.multiple_of(j * ch, ch)
            pltpu.async_copy(
                table_hbm.at[idx_v.at[pl.ds(off, ch)]], rows_v, sem).wait()
            pltpu.sync_copy(rows_v, out_hbm.at[pl.ds(base + off, ch)])
            return carry

        lax.fori_loop(0, n_ch, body, 0)

    return k


def _transpose(x, br, bc):
    n, m = x.shape

    def body(x_ref, o_ref):
        o_ref[...] = jnp.transpose(x_ref[...], (1, 0))

    return pl.pallas_call(
        body,
        grid=(n // br, m // bc),
        in_specs=[pl.BlockSpec((br, bc), lambda i, j: (i, j))],
        out_specs=pl.BlockSpec((bc, br), lambda i, j: (j, i)),
        out_shape=jax.ShapeDtypeStruct((m, n), x.dtype),
    )(x)


def kernel(inputs, coordinates):
    b = inputs.shape[0]
    c = coordinates[:, :, 0]
    sq = jnp.sum(c * c, axis=0)
    d, blk = _dist_and_blocks(c, sq)

    rows = jnp.arange(F, dtype=jnp.int32)[:, None]
    flat_ridx = (rows * NB + blk).reshape(-1)
    cand = _make_gather(F * NB, BW, F * KB, jnp.float32)(
        d.reshape(F * NB, BW), flat_ridx)

    idx = _select_topk(cand.reshape(F, W), blk)

    data = _transpose(inputs[:, :, 0], 64, 2048)
    gathered = _make_gather(F, b, F * K, jnp.float32)(
        data, idx.reshape(-1))
    out = _transpose(gathered, 2048, 64)
    return out[:, :, None]


# speedup vs baseline: 9.0959x; 1.0639x over previous
import functools

import jax
import jax.numpy as jnp
from jax import lax
from jax.experimental import pallas as pl
from jax.experimental.pallas import tpu as pltpu
from jax.experimental.pallas import tpu_sc as plsc

F = 8192
D = 256
K = 16
R = 256
BW = 32
NB = F // BW
KB = 20
W = KB * BW


def _dist_blocks_body(sqc_ref, sqr_ref, call_ref, crows_ref, d_ref, blk_ref):
    c = lax.dot_general(crows_ref[...], call_ref[...],
                        (((0,), (0,)), ((), ())))
    d = (sqc_ref[...] + sqr_ref[...]) - 2.0 * c
    d = jnp.maximum(d, 0.0)
    d_ref[...] = d

    dt = jnp.transpose(d, (1, 0))
    bmin = jnp.min(dt.reshape(NB, BW, R), axis=1)
    iota_b = lax.broadcasted_iota(jnp.int32, (NB, R), 0)
    row_k = lax.broadcasted_iota(jnp.int32, (KB, R), 0)

    def body(j, carry):
        bb, acc = carry
        m = jnp.min(bb, axis=0, keepdims=True)
        cand = jnp.where(bb == m, iota_b, jnp.int32(NB))
        amin = jnp.min(cand, axis=0, keepdims=True)
        acc = jnp.where(row_k == j, amin, acc)
        bb = jnp.where(iota_b == amin, jnp.float32(jnp.inf), bb)
        return bb, acc

    _, blk_t = lax.fori_loop(0, KB, body,
                             (bmin, jnp.zeros((KB, R), jnp.int32)))
    blk_ref[...] = jnp.transpose(blk_t, (1, 0))


def _dist_and_blocks(c, sq, h, nh):
    hh = F // nh
    t0 = h * (hh // R)
    return pl.pallas_call(
        _dist_blocks_body,
        grid=(hh // R,),
        in_specs=[
            pl.BlockSpec((1, F), lambda i: (0, 0)),
            pl.BlockSpec((R, 1), lambda i: (i, 0)),
            pl.BlockSpec((D, F), lambda i: (0, 0)),
            pl.BlockSpec((D, R), lambda i: (0, i + t0)),
        ],
        out_specs=[
            pl.BlockSpec((R, F), lambda i: (i, 0)),
            pl.BlockSpec((R, KB), lambda i: (i, 0)),
        ],
        out_shape=[
            jax.ShapeDtypeStruct((hh, F), jnp.float32),
            jax.ShapeDtypeStruct((hh, KB), jnp.int32),
        ],
    )(sq[None, :], sq[h * hh:(h + 1) * hh, None], c, c)


def _select_body(cand_ref, blk_ref, idx_ref):
    dd0 = cand_ref[...]
    blk = blk_ref[...]
    off = lax.broadcasted_iota(jnp.int32, (R, KB, BW), 2)
    gg = (blk[:, :, None] * BW + off).reshape(R, W)
    colk = lax.broadcasted_iota(jnp.int32, (R, K), 1)

    def body(j, carry):
        dd, acc = carry
        m = jnp.min(dd, axis=1, keepdims=True)
        ag = jnp.where(dd == m, gg, jnp.int32(F))
        amin = jnp.min(ag, axis=1, keepdims=True)
        acc = jnp.where(colk == j, amin, acc)
        dd = jnp.where(gg == amin, jnp.float32(jnp.inf), dd)
        return dd, acc

    _, acc = lax.fori_loop(0, K, body, (dd0, jnp.zeros((R, K), jnp.int32)))
    idx_ref[...] = acc


def _select_topk(cand, blk):
    n = cand.shape[0]
    return pl.pallas_call(
        _select_body,
        grid=(n // R,),
        in_specs=[
            pl.BlockSpec((R, W), lambda i: (i, 0)),
            pl.BlockSpec((R, KB), lambda i: (i, 0)),
        ],
        out_specs=pl.BlockSpec((R, K), lambda i: (i, 0)),
        out_shape=jax.ShapeDtypeStruct((n, K), jnp.int32),
    )(cand, blk)


def _make_gather(n_rows, n_cols, n_idx, dtype):
    info = plsc.get_sparse_core_info()
    nw = info.num_cores * info.num_subcores
    per_w = n_idx // nw
    ch = 128
    n_ch = per_w // ch
    mesh = plsc.VectorSubcoreMesh(core_axis_name="c", subcore_axis_name="s")

    @functools.partial(
        pl.kernel, mesh=mesh,
        compiler_params=pltpu.CompilerParams(use_tc_tiling_on_sc=False),
        out_type=jax.ShapeDtypeStruct((n_idx, n_cols), dtype),
        scratch_types=[
            pltpu.VMEM((per_w,), jnp.int32),
            pltpu.VMEM((ch, n_cols), dtype),
            pltpu.SemaphoreType.DMA,
        ],
    )
    def k(table_hbm, idx_hbm, out_hbm, idx_v, rows_v, sem):
        wid = lax.axis_index("s") * info.num_cores + lax.axis_index("c")
        base = wid * per_w
        pltpu.sync_copy(idx_hbm.at[pl.ds(base, per_w)], idx_v)

        def body(j, carry):
            off = pl.multiple_of(j * ch, ch)
            pltpu.async_copy(
                table_hbm.at[idx_v.at[pl.ds(off, ch)]], rows_v, sem).wait()
            pltpu.sync_copy(rows_v, out_hbm.at[pl.ds(base + off, ch)])
            return carry

        lax.fori_loop(0, n_ch, body, 0)

    return k


def _transpose(x, br, bc):
    n, m = x.shape

    def body(x_ref, o_ref):
        o_ref[...] = jnp.transpose(x_ref[...], (1, 0))

    return pl.pallas_call(
        body,
        grid=(n // br, m // bc),
        in_specs=[pl.BlockSpec((br, bc), lambda i, j: (i, j))],
        out_specs=pl.BlockSpec((bc, br), lambda i, j: (j, i)),
        out_shape=jax.ShapeDtypeStruct((m, n), x.dtype),
    )(x)


def kernel(inputs, coordinates):
    b = inputs.shape[0]
    c = coordinates[:, :, 0]
    sq = jnp.sum(c * c, axis=0)

    nh = 2
    hh = F // nh
    rows = jnp.arange(hh, dtype=jnp.int32)[:, None]
    halves = [_dist_and_blocks(c, sq, h, nh) for h in range(nh)]
    cands = []
    for d, blk in halves:
        flat_ridx = (rows * NB + blk).reshape(-1)
        cands.append(_make_gather(hh * NB, BW, hh * KB, jnp.float32)(
            d.reshape(hh * NB, BW), flat_ridx))
    idx = jnp.concatenate(
        [_select_topk(cand.reshape(hh, W), blk)
         for cand, (_, blk) in zip(cands, halves)], axis=0)

    data = _transpose(inputs[:, :, 0], 64, 2048)
    gathered = _make_gather(F, b, F * K, jnp.float32)(
        data, idx.reshape(-1))
    out = _transpose(gathered, 2048, 64)
    return out[:, :, None]


# speedup vs baseline: 10.6437x; 1.1702x over previous
import functools

import jax
import jax.numpy as jnp
from jax import lax
from jax.experimental import pallas as pl
from jax.experimental.pallas import tpu as pltpu
from jax.experimental.pallas import tpu_sc as plsc

F = 8192
D = 256
K = 16
R = 256
BW = 32
NB = F // BW
KB = 20
W = KB * BW


NBW = F // 128


def _dist_blocks_body(sqc_ref, sqr_ref, call_ref, crows_ref, d_ref, blk_ref):
    c = lax.dot_general(crows_ref[...], call_ref[...],
                        (((0,), (0,)), ((), ())))
    d = (sqc_ref[...] + sqr_ref[...]) - 2.0 * c
    d = jnp.maximum(d, 0.0)
    for bb in range(NBW):
        d_ref[bb, :, :] = d[:, 128 * bb:128 * (bb + 1)]

    dt = jnp.transpose(d, (1, 0))
    bmin = jnp.min(dt.reshape(NB, BW, R), axis=1)
    iota_b = lax.broadcasted_iota(jnp.int32, (NB, R), 0)
    row_k = lax.broadcasted_iota(jnp.int32, (KB, R), 0)

    def body(j, carry):
        bb, acc = carry
        m = jnp.min(bb, axis=0, keepdims=True)
        cand = jnp.where(bb == m, iota_b, jnp.int32(NB))
        amin = jnp.min(cand, axis=0, keepdims=True)
        acc = jnp.where(row_k == j, amin, acc)
        bb = jnp.where(iota_b == amin, jnp.float32(jnp.inf), bb)
        return bb, acc

    _, blk_t = lax.fori_loop(0, KB, body,
                             (bmin, jnp.zeros((KB, R), jnp.int32)))
    blk_ref[...] = jnp.transpose(blk_t, (1, 0))


def _dist_and_blocks(c, sq, h, nh):
    hh = F // nh
    t0 = h * (hh // R)
    return pl.pallas_call(
        _dist_blocks_body,
        grid=(hh // R,),
        in_specs=[
            pl.BlockSpec((1, F), lambda i: (0, 0)),
            pl.BlockSpec((R, 1), lambda i: (i, 0)),
            pl.BlockSpec((D, F), lambda i: (0, 0)),
            pl.BlockSpec((D, R), lambda i: (0, i + t0)),
        ],
        out_specs=[
            pl.BlockSpec((NBW, R, 128), lambda i: (0, i, 0)),
            pl.BlockSpec((R, KB), lambda i: (i, 0)),
        ],
        out_shape=[
            jax.ShapeDtypeStruct((NBW, hh, 128), jnp.float32),
            jax.ShapeDtypeStruct((hh, KB), jnp.int32),
        ],
    )(sq[None, :], sq[h * hh:(h + 1) * hh, None], c, c)


def _select_body(cand_ref, blk_ref, idx_ref):
    dd0 = cand_ref[...]
    blk = blk_ref[...]
    off = lax.broadcasted_iota(jnp.int32, (R, KB, BW), 2)
    gg = (blk[:, :, None] * BW + off).reshape(R, W)
    colk = lax.broadcasted_iota(jnp.int32, (R, K), 1)

    def body(j, carry):
        dd, acc = carry
        m = jnp.min(dd, axis=1, keepdims=True)
        ag = jnp.where(dd == m, gg, jnp.int32(F))
        amin = jnp.min(ag, axis=1, keepdims=True)
        acc = jnp.where(colk == j, amin, acc)
        dd = jnp.where(gg == amin, jnp.float32(jnp.inf), dd)
        return dd, acc

    _, acc = lax.fori_loop(0, K, body, (dd0, jnp.zeros((R, K), jnp.int32)))
    idx_ref[...] = acc


def _select_topk(cand, blk):
    n = cand.shape[0]
    return pl.pallas_call(
        _select_body,
        grid=(n // R,),
        in_specs=[
            pl.BlockSpec((R, W), lambda i: (i, 0)),
            pl.BlockSpec((R, KB), lambda i: (i, 0)),
        ],
        out_specs=pl.BlockSpec((R, K), lambda i: (i, 0)),
        out_shape=jax.ShapeDtypeStruct((n, K), jnp.int32),
    )(cand, blk)


def _make_gather(n_rows, n_cols, n_idx, dtype):
    info = plsc.get_sparse_core_info()
    nw = info.num_cores * info.num_subcores
    per_w = n_idx // nw
    ch = 128
    n_ch = per_w // ch
    mesh = plsc.VectorSubcoreMesh(core_axis_name="c", subcore_axis_name="s")

    @functools.partial(
        pl.kernel, mesh=mesh,
        compiler_params=pltpu.CompilerParams(use_tc_tiling_on_sc=False),
        out_type=jax.ShapeDtypeStruct((n_idx, n_cols), dtype),
        scratch_types=[
            pltpu.VMEM((per_w,), jnp.int32),
            pltpu.VMEM((ch, n_cols), dtype),
            pltpu.SemaphoreType.DMA,
        ],
    )
    def k(table_hbm, idx_hbm, out_hbm, idx_v, rows_v, sem):
        wid = lax.axis_index("s") * info.num_cores + lax.axis_index("c")
        base = wid * per_w
        pltpu.sync_copy(idx_hbm.at[pl.ds(base, per_w)], idx_v)

        def body(j, carry):
            off = pl.multiple_of(j * ch, ch)
            pltpu.async_copy(
                table_hbm.at[idx_v.at[pl.ds(off, ch)]], rows_v, sem).wait()
            pltpu.sync_copy(rows_v, out_hbm.at[pl.ds(base + off, ch)])
            return carry

        lax.fori_loop(0, n_ch, body, 0)

    return k


def _transpose(x, br, bc):
    n, m = x.shape

    def body(x_ref, o_ref):
        o_ref[...] = jnp.transpose(x_ref[...], (1, 0))

    return pl.pallas_call(
        body,
        grid=(n // br, m // bc),
        in_specs=[pl.BlockSpec((br, bc), lambda i, j: (i, j))],
        out_specs=pl.BlockSpec((bc, br), lambda i, j: (j, i)),
        out_shape=jax.ShapeDtypeStruct((m, n), x.dtype),
    )(x)


def kernel(inputs, coordinates):
    b = inputs.shape[0]
    c = coordinates[:, :, 0]
    sq = jnp.sum(c * c, axis=0)

    nh = 2
    hh = F // nh
    rows = jnp.arange(hh, dtype=jnp.int32)[:, None]
    halves = [_dist_and_blocks(c, sq, h, nh) for h in range(nh)]
    cands = []
    for d, blk in halves:
        flat_ridx = ((blk >> 2) * (4 * hh) + 4 * rows
                     + (blk & 3)).reshape(-1)
        cands.append(_make_gather(hh * NB, BW, hh * KB, jnp.float32)(
            d.reshape(hh * NB, BW), flat_ridx))
    idx = jnp.concatenate(
        [_select_topk(cand.reshape(hh, W), blk)
         for cand, (_, blk) in zip(cands, halves)], axis=0)

    data = _transpose(inputs[:, :, 0], 64, 2048)
    gathered = _make_gather(F, b, F * K, jnp.float32)(
        data, idx.reshape(-1))
    out = _transpose(gathered, 2048, 64)
    return out[:, :, None]
